# baseline (speedup 1.0000x reference)
import functools

import jax
import jax.numpy as jnp
from jax import lax
from jax.experimental import pallas as pl
from jax.experimental.pallas import tpu as pltpu
from jax.experimental.pallas import tpu_sc as plsc

N_NODES = 10000
D_FEAT = 128
HIDDEN = 64
NEG_SLOPE = 0.01
N_EDGES = 320000

NC = 2
NS = 16
L = 16
NW = NC * NS

CHUNK = 128
CH_PER_W = 80
E_PAD = NW * CH_PER_W * CHUNK
N_PAD = 10112
ROWS_PER_TILE = N_PAD // NS
HALF = N_PAD // 2
NV = CH_PER_W * CHUNK // L
HV = HALF // L

_MESH = plsc.VectorSubcoreMesh(core_axis_name="c", subcore_axis_name="s")
_SC_PARAMS = pltpu.CompilerParams(use_tc_tiling_on_sc=False,
                                  needs_layout_passes=False)


def _worker_ids():
    cid = lax.axis_index("c")
    sid = lax.axis_index("s")
    return cid, sid, sid * NC + cid


def _deg_body(dst_hbm, out_hbm, dst_v, red_v):
    cid, sid, wid = _worker_ids()
    pltpu.sync_copy(dst_hbm.at[wid], dst_v)
    ones16 = jnp.full((L,), 1.0, jnp.float32)
    zeros16 = jnp.zeros((L,), jnp.float32)

    def zb(i, c):
        red_v[pl.ds(i * L, L)] = zeros16
        return c

    lax.fori_loop(0, N_PAD // L, zb, 0)

    def cb(i, c):
        idx = dst_v[i // 8, pl.ds((i % 8) * L, L)]
        plsc.addupdate_scatter(red_v, [idx], ones16)
        return c

    lax.fori_loop(0, NV, cb, 0)
    pltpu.sync_copy(red_v, out_hbm.at[wid])


_deg = functools.partial(
    pl.kernel,
    out_type=jax.ShapeDtypeStruct((NW, N_PAD), jnp.float32),
    mesh=_MESH,
    compiler_params=_SC_PARAMS,
    scratch_types=[
        pltpu.VMEM((CH_PER_W, CHUNK), jnp.int32),
        pltpu.VMEM((N_PAD,), jnp.float32),
    ],
)(_deg_body)


NBUF = 4


def _agg_body(u_hbm, src_hbm, dst_hbm, zeros_hbm, out_hbm,
              src_v, dst_v, rows_v, agg_sh,
              zsem, g0, g1, g2, g3, s0, s1, s2, s3):
    cid, sid, wid = _worker_ids()
    base = sid * ROWS_PER_TILE
    pltpu.sync_copy(src_hbm.at[wid], src_v)
    pltpu.sync_copy(dst_hbm.at[wid], dst_v)
    pltpu.async_copy(zeros_hbm.at[pl.ds(base, ROWS_PER_TILE)],
                     agg_sh.at[pl.ds(base, ROWS_PER_TILE)], zsem)
    gs = (g0, g1, g2, g3)
    ss = (s0, s1, s2, s3)
    for b in range(NBUF - 1):
        pltpu.async_copy(u_hbm.at[src_v.at[b]], rows_v.at[b], gs[b])
    pltpu.make_async_copy(zeros_hbm.at[pl.ds(base, ROWS_PER_TILE)],
                          agg_sh.at[pl.ds(base, ROWS_PER_TILE)], zsem).wait()
    plsc.subcore_barrier()

    def body(g, carry):
        for b in range(NBUF):
            j = g * NBUF + b
            bb = (b + NBUF - 1) % NBUF

            @pl.when((j >= 1) & (j + NBUF - 1 < CH_PER_W))
            def _():
                pltpu.make_async_copy(rows_v.at[bb],
                                      agg_sh.at[dst_v.at[j - 1]],
                                      ss[bb]).wait()
                pltpu.async_copy(u_hbm.at[src_v.at[j + NBUF - 1]],
                                 rows_v.at[bb], gs[bb])

            @pl.when(j == 0)
            def _():
                pltpu.async_copy(u_hbm.at[src_v.at[NBUF - 1]],
                                 rows_v.at[NBUF - 1], gs[NBUF - 1])

            pltpu.make_async_copy(u_hbm.at[src_v.at[j]], rows_v.at[b],
                                  gs[b]).wait()
            pltpu.async_copy(rows_v.at[b], agg_sh.at[dst_v.at[j]], ss[b],
                             add=True)
        return carry

    lax.fori_loop(0, CH_PER_W // NBUF, body, 0)
    for b in range(NBUF):
        pltpu.make_async_copy(rows_v.at[b],
                              agg_sh.at[dst_v.at[CH_PER_W - NBUF + b]],
                              ss[b]).wait()
    plsc.subcore_barrier()
    pltpu.sync_copy(agg_sh.at[pl.ds(base, ROWS_PER_TILE)],
                    out_hbm.at[cid, pl.ds(base, ROWS_PER_TILE)])


_agg = functools.partial(
    pl.kernel,
    out_type=jax.ShapeDtypeStruct((NC, N_PAD, HIDDEN), jnp.float32),
    mesh=_MESH,
    compiler_params=_SC_PARAMS,
    scratch_types=[
        pltpu.VMEM((CH_PER_W, CHUNK), jnp.int32),
        pltpu.VMEM((CH_PER_W, CHUNK), jnp.int32),
        pltpu.VMEM((NBUF, CHUNK, HIDDEN), jnp.float32),
        pltpu.VMEM_SHARED((N_PAD, HIDDEN), jnp.float32),
        pltpu.SemaphoreType.DMA,
        pltpu.SemaphoreType.DMA,
        pltpu.SemaphoreType.DMA,
        pltpu.SemaphoreType.DMA,
        pltpu.SemaphoreType.DMA,
        pltpu.SemaphoreType.DMA,
        pltpu.SemaphoreType.DMA,
        pltpu.SemaphoreType.DMA,
        pltpu.SemaphoreType.DMA,
    ],
)(_agg_body)


def _tc_xw_body(x_ref, w_ref, h_ref):
    h_ref[...] = jnp.dot(x_ref[...], w_ref[...],
                         preferred_element_type=jnp.float32)


_tc_xw = pl.pallas_call(
    _tc_xw_body,
    out_shape=jax.ShapeDtypeStruct((N_PAD, HIDDEN), jnp.float32),
)


def _tc_scale_body(deg_ref, ones_ref, h_ref, u_ref, dinv_ref):
    deg = lax.dot_general(deg_ref[...], ones_ref[...],
                          (((0,), (0,)), ((), ())),
                          preferred_element_type=jnp.float32)
    dinv = lax.rsqrt(deg + 1.0)
    u_ref[...] = h_ref[...] * dinv
    dinv_ref[...] = dinv


_tc_scale = pl.pallas_call(
    _tc_scale_body,
    out_shape=[
        jax.ShapeDtypeStruct((N_PAD, HIDDEN), jnp.float32),
        jax.ShapeDtypeStruct((N_PAD, 1), jnp.float32),
    ],
)


def _tc_mid_body(agg_ref, u_ref, dinv_ref, b_ref, w_ref, unext_ref):
    a = agg_ref[...]
    dinv = dinv_ref[...]
    z = (a[0] + a[1] + u_ref[...]) * dinv + b_ref[...]
    h = jnp.where(z >= 0, z, NEG_SLOPE * z)
    unext_ref[...] = jnp.dot(h, w_ref[...],
                             preferred_element_type=jnp.float32) * dinv


_tc_mid = pl.pallas_call(
    _tc_mid_body,
    out_shape=jax.ShapeDtypeStruct((N_PAD, HIDDEN), jnp.float32),
)


def _tc_last_body(agg_ref, u_ref, dinv_ref, b_ref, out_ref):
    nd = pl.ds(0, N_NODES)
    a0 = agg_ref[0, nd, :]
    a1 = agg_ref[1, nd, :]
    z = (a0 + a1 + u_ref[nd, :]) * dinv_ref[nd, :] + b_ref[...]
    out_ref[...] = jnp.where(z >= 0, z, NEG_SLOPE * z)


_tc_last = pl.pallas_call(
    _tc_last_body,
    out_shape=jax.ShapeDtypeStruct((N_NODES, HIDDEN), jnp.float32),
)


def kernel(x, edge_index, W0, b0, W1, b1, W2, b2):
    src = edge_index[0]
    dst = edge_index[1]
    pad_per_w = (E_PAD - N_EDGES) // NW
    real_per_w = N_EDGES // NW
    pad_ids = (jnp.arange(E_PAD - N_EDGES, dtype=jnp.int32)
               % (N_PAD - N_NODES)) + N_NODES
    pad_block = pad_ids.reshape(NW, pad_per_w)
    src_p = jnp.concatenate([src.reshape(NW, real_per_w), pad_block],
                            axis=1).reshape(NW, CH_PER_W, CHUNK)
    dst_p = jnp.concatenate([dst.reshape(NW, real_per_w), pad_block],
                            axis=1).reshape(NW, CH_PER_W, CHUNK)
    x_p = jnp.pad(x, ((0, N_PAD - N_NODES), (0, 0)))

    ones_nw = jnp.ones((NW, 1), jnp.float32)
    zeros_h = jnp.zeros((N_PAD, HIDDEN), jnp.float32)
    b0r = b0.reshape(1, HIDDEN)
    b1r = b1.reshape(1, HIDDEN)
    b2r = b2.reshape(1, HIDDEN)

    deg_parts = _deg(dst_p)
    h0 = _tc_xw(x_p, W0)
    u0, dinv = _tc_scale(deg_parts, ones_nw, h0)
    agg0 = _agg(u0, src_p, dst_p, zeros_h)
    u1 = _tc_mid(agg0, u0, dinv, b0r, W1)
    agg1 = _agg(u1, src_p, dst_p, zeros_h)
    u2 = _tc_mid(agg1, u1, dinv, b1r, W2)
    agg2 = _agg(u2, src_p, dst_p, zeros_h)
    return _tc_last(agg2, u2, dinv, b2r)


# speedup vs baseline: 1.2285x; 1.2285x over previous
import functools

import jax
import jax.numpy as jnp
from jax import lax
from jax.experimental import pallas as pl
from jax.experimental.pallas import tpu as pltpu
from jax.experimental.pallas import tpu_sc as plsc

N_NODES = 10000
D_FEAT = 128
HIDDEN = 64
NEG_SLOPE = 0.01
N_EDGES = 320000

NC = 2
NS = 16
L = 16
NW = NC * NS

CHUNK = 128
CH_PER_W = 80
E_PAD = NW * CH_PER_W * CHUNK
N_PAD = 10240
NP2 = N_PAD // 2
PK = 2 * HIDDEN
ROWS_PER_TILE = N_PAD // NS
NV = CH_PER_W * CHUNK // L

_MESH = plsc.VectorSubcoreMesh(core_axis_name="c", subcore_axis_name="s")
_SC_PARAMS = pltpu.CompilerParams(use_tc_tiling_on_sc=False,
                                  needs_layout_passes=False)


def _worker_ids():
    cid = lax.axis_index("c")
    sid = lax.axis_index("s")
    return cid, sid, sid * NC + cid


def _deg_body(dst_hbm, out_hbm, dst_v, red_v):
    cid, sid, wid = _worker_ids()
    pltpu.sync_copy(dst_hbm.at[wid], dst_v)
    ones16 = jnp.full((L,), 1.0, jnp.float32)
    zeros16 = jnp.zeros((L,), jnp.float32)

    def zb(i, c):
        red_v[pl.ds(i * L, L)] = zeros16
        return c

    lax.fori_loop(0, N_PAD // L, zb, 0)

    def cb(i, c):
        idx = dst_v[i // 8, pl.ds((i % 8) * L, L)]
        addr = (idx & 1) * NP2 + (idx >> 1)
        plsc.addupdate_scatter(red_v, [addr], ones16)
        return c

    lax.fori_loop(0, NV, cb, 0)
    pltpu.sync_copy(red_v, out_hbm.at[wid])


_deg = functools.partial(
    pl.kernel,
    out_type=jax.ShapeDtypeStruct((NW, N_PAD), jnp.float32),
    mesh=_MESH,
    compiler_params=_SC_PARAMS,
    scratch_types=[
        pltpu.VMEM((CH_PER_W, CHUNK), jnp.int32),
        pltpu.VMEM((N_PAD,), jnp.float32),
    ],
)(_deg_body)


NBUF = 4


def _agg_body(u_hbm, src_hbm, dst_hbm, zeros_hbm, out_hbm,
              src_v, dst_v, rows_v, agg_sh,
              zsem, g0, g1, g2, g3, s0, s1, s2, s3):
    cid, sid, wid = _worker_ids()
    base = sid * ROWS_PER_TILE
    pltpu.sync_copy(src_hbm.at[wid], src_v)
    pltpu.sync_copy(dst_hbm.at[wid], dst_v)
    pltpu.async_copy(zeros_hbm.at[pl.ds(base, ROWS_PER_TILE)],
                     agg_sh.at[pl.ds(base, ROWS_PER_TILE)], zsem)
    gs = (g0, g1, g2, g3)
    ss = (s0, s1, s2, s3)
    for b in range(NBUF - 1):
        pltpu.async_copy(u_hbm.at[src_v.at[b]], rows_v.at[b], gs[b])
    pltpu.make_async_copy(zeros_hbm.at[pl.ds(base, ROWS_PER_TILE)],
                          agg_sh.at[pl.ds(base, ROWS_PER_TILE)], zsem).wait()
    plsc.subcore_barrier()

    def body(g, carry):
        for b in range(NBUF):
            j = g * NBUF + b
            bb = (b + NBUF - 1) % NBUF

            @pl.when((j >= 1) & (j + NBUF - 1 < CH_PER_W))
            def _():
                pltpu.make_async_copy(rows_v.at[bb],
                                      agg_sh.at[dst_v.at[j - 1]],
                                      ss[bb]).wait()
                pltpu.async_copy(u_hbm.at[src_v.at[j + NBUF - 1]],
                                 rows_v.at[bb], gs[bb])

            @pl.when(j == 0)
            def _():
                pltpu.async_copy(u_hbm.at[src_v.at[NBUF - 1]],
                                 rows_v.at[NBUF - 1], gs[NBUF - 1])

            pltpu.make_async_copy(u_hbm.at[src_v.at[j]], rows_v.at[b],
                                  gs[b]).wait()
            pltpu.async_copy(rows_v.at[b], agg_sh.at[dst_v.at[j]], ss[b],
                             add=True)
        return carry

    lax.fori_loop(0, CH_PER_W // NBUF, body, 0)
    for b in range(NBUF):
        pltpu.make_async_copy(rows_v.at[b],
                              agg_sh.at[dst_v.at[CH_PER_W - NBUF + b]],
                              ss[b]).wait()
    plsc.subcore_barrier()
    pltpu.sync_copy(agg_sh.at[pl.ds(base, ROWS_PER_TILE)],
                    out_hbm.at[cid, pl.ds(base, ROWS_PER_TILE)])


_agg = functools.partial(
    pl.kernel,
    out_type=jax.ShapeDtypeStruct((NC, N_PAD, HIDDEN), jnp.float32),
    mesh=_MESH,
    compiler_params=_SC_PARAMS,
    scratch_types=[
        pltpu.VMEM((CH_PER_W, CHUNK), jnp.int32),
        pltpu.VMEM((CH_PER_W, CHUNK), jnp.int32),
        pltpu.VMEM((NBUF, CHUNK, HIDDEN), jnp.float32),
        pltpu.VMEM_SHARED((N_PAD, HIDDEN), jnp.float32),
        pltpu.SemaphoreType.DMA,
        pltpu.SemaphoreType.DMA,
        pltpu.SemaphoreType.DMA,
        pltpu.SemaphoreType.DMA,
        pltpu.SemaphoreType.DMA,
        pltpu.SemaphoreType.DMA,
        pltpu.SemaphoreType.DMA,
        pltpu.SemaphoreType.DMA,
        pltpu.SemaphoreType.DMA,
    ],
)(_agg_body)


def _tc_xw_body(x2_ref, w2_ref, h2_ref):
    h2_ref[...] = jnp.dot(x2_ref[...], w2_ref[...],
                          preferred_element_type=jnp.float32)


_tc_xw = pl.pallas_call(
    _tc_xw_body,
    out_shape=jax.ShapeDtypeStruct((NP2, PK), jnp.float32),
)


def _tc_scale_body(degp_ref, bsel_ref, h2_ref, u2_ref, dinv2_ref):
    deg2 = lax.dot_general(degp_ref[...], bsel_ref[...],
                           (((0,), (0,)), ((), ())),
                           preferred_element_type=jnp.float32)
    dinv2 = lax.rsqrt(deg2 + 1.0)
    u2_ref[...] = h2_ref[...] * dinv2
    dinv2_ref[...] = dinv2


_tc_scale = pl.pallas_call(
    _tc_scale_body,
    out_shape=[
        jax.ShapeDtypeStruct((NP2, PK), jnp.float32),
        jax.ShapeDtypeStruct((NP2, PK), jnp.float32),
    ],
)


def _tc_mid_body(agg_ref, u2_ref, dinv2_ref, b2_ref, w2_ref, unext_ref):
    a = agg_ref[...]
    dinv2 = dinv2_ref[...]
    z = (a[0] + a[1] + u2_ref[...]) * dinv2 + b2_ref[...]
    h = jnp.where(z >= 0, z, NEG_SLOPE * z)
    unext_ref[...] = jnp.dot(h, w2_ref[...],
                             preferred_element_type=jnp.float32) * dinv2


_tc_mid = pl.pallas_call(
    _tc_mid_body,
    out_shape=jax.ShapeDtypeStruct((NP2, PK), jnp.float32),
)


def _tc_last_body(agg_ref, u2_ref, dinv2_ref, b2_ref, out_ref):
    a = agg_ref[...]
    z = (a[0] + a[1] + u2_ref[...]) * dinv2_ref[...] + b2_ref[...]
    out_ref[...] = jnp.where(z >= 0, z, NEG_SLOPE * z)


_tc_last = pl.pallas_call(
    _tc_last_body,
    out_shape=jax.ShapeDtypeStruct((NP2, PK), jnp.float32),
)


def _blkdiag(w):
    z = jnp.zeros_like(w)
    return jnp.concatenate(
        [jnp.concatenate([w, z], axis=1), jnp.concatenate([z, w], axis=1)],
        axis=0)


def kernel(x, edge_index, W0, b0, W1, b1, W2, b2):
    src = edge_index[0]
    dst = edge_index[1]
    pad_per_w = (E_PAD - N_EDGES) // NW
    real_per_w = N_EDGES // NW
    pad_ids = (jnp.arange(E_PAD - N_EDGES, dtype=jnp.int32)
               % (N_PAD - N_NODES)) + N_NODES
    pad_block = pad_ids.reshape(NW, pad_per_w)
    src_p = jnp.concatenate([src.reshape(NW, real_per_w), pad_block],
                            axis=1).reshape(NW, CH_PER_W, CHUNK)
    dst_p = jnp.concatenate([dst.reshape(NW, real_per_w), pad_block],
                            axis=1).reshape(NW, CH_PER_W, CHUNK)
    x2 = jnp.pad(x, ((0, N_PAD - N_NODES), (0, 0))).reshape(NP2, 2 * D_FEAT)

    half = jnp.concatenate([jnp.ones((HIDDEN,), jnp.float32),
                            jnp.zeros((HIDDEN,), jnp.float32)])
    bsel = jnp.stack([half, 1.0 - half])
    bsel = jnp.tile(bsel, (NW, 1))

    zeros_h = jnp.zeros((N_PAD, HIDDEN), jnp.float32)
    b0r = jnp.concatenate([b0, b0]).reshape(1, PK)
    b1r = jnp.concatenate([b1, b1]).reshape(1, PK)
    b2r = jnp.concatenate([b2, b2]).reshape(1, PK)
    W0b = _blkdiag(W0)
    W1b = _blkdiag(W1)
    W2b = _blkdiag(W2)

    deg_parts = _deg(dst_p)
    h0 = _tc_xw(x2, W0b)
    u0p, dinv2 = _tc_scale(deg_parts.reshape(NW * 2, NP2), bsel, h0)
    agg0 = _agg(u0p.reshape(N_PAD, HIDDEN), src_p, dst_p, zeros_h)
    u1p = _tc_mid(agg0.reshape(NC, NP2, PK), u0p, dinv2, b0r, W1b)
    agg1 = _agg(u1p.reshape(N_PAD, HIDDEN), src_p, dst_p, zeros_h)
    u2p = _tc_mid(agg1.reshape(NC, NP2, PK), u1p, dinv2, b1r, W2b)
    agg2 = _agg(u2p.reshape(N_PAD, HIDDEN), src_p, dst_p, zeros_h)
    out2 = _tc_last(agg2.reshape(NC, NP2, PK), u2p, dinv2, b2r)
    return out2.reshape(N_PAD, HIDDEN)[:N_NODES]
